# Initial kernel scaffold; baseline (speedup 1.0000x reference)
import jax
import jax.numpy as jnp
from jax.experimental import pallas as pl


def kernel(x, edge_index, W_pre, b_pre, W1_0, b1_0, W2_0, b2_0, W1_1, b1_1, W2_1, b2_1, W_head, b_head):
    raise NotImplementedError("write your pallas kernel here")


# speedup vs baseline: 4.9836x; 4.9836x over previous
import functools

import jax
import jax.numpy as jnp
from jax import lax
from jax.experimental import pallas as pl
from jax.experimental.pallas import tpu as pltpu
from jax.experimental.pallas import tpu_sc as plsc

N_REAL = 10000
NP = 10240
HALF = NP // 2
ACCR = HALF + 128
DIM = 256
FH = 128
NC, NS = 2, 16
EPW = 10240
NCHUNK, CHUNK = 80, 128
PCH = 1296
PSLOT = PCH * CHUNK
FILLCH = PCH // 8
MAXCH = PCH // NS
RPT = NP // NS
ALPHA = 0.1
K_ITER = 10
ROWBLK = 1280
GRID = NP // ROWBLK

_mesh1 = plsc.VectorSubcoreMesh(core_axis_name="c", subcore_axis_name="s",
                                num_cores=1)
_mesh2 = plsc.VectorSubcoreMesh(core_axis_name="c", subcore_axis_name="s",
                                num_cores=2)


def _fill_pattern(buf, base):
    def body(i, _):
        for k in range(8):
            buf[pl.ds(i * 128 + k * 16, 16)] = (lax.iota(jnp.int32, 16)
                                                + (base + k * 16))
        return 0

    lax.fori_loop(0, 16384 // 128, body, 0)


@functools.partial(
    pl.kernel,
    out_type=[
        jax.ShapeDtypeStruct((NP,), jnp.float32),
        jax.ShapeDtypeStruct((2 * PSLOT,), jnp.int32),
        jax.ShapeDtypeStruct((2 * PSLOT,), jnp.int32),
        jax.ShapeDtypeStruct((16,), jnp.int32),
    ],
    mesh=_mesh1,
    compiler_params=pltpu.CompilerParams(needs_layout_passes=False),
    scratch_types=[
        pltpu.VMEM((NCHUNK, CHUNK), jnp.int32),
        pltpu.VMEM((NCHUNK, CHUNK), jnp.int32),
        pltpu.VMEM((NCHUNK, CHUNK), jnp.int32),
        pltpu.VMEM((NCHUNK, CHUNK), jnp.int32),
        pltpu.VMEM((16384,), jnp.int32),
        pltpu.VMEM((CHUNK,), jnp.float32),
        pltpu.VMEM((1024,), jnp.float32),
        pltpu.VMEM((16,), jnp.int32),
        pltpu.VMEM((8, 128), jnp.int32),
        pltpu.VMEM_SHARED((NS, 8, 128), jnp.int32),
        pltpu.VMEM_SHARED((NP,), jnp.float32),
    ],
)
def _sc_prep(src_hbm, dst_hbm, deg_out, srcp_out, dstp_out, cnt_out,
             src_v, dst_v, dl_v, pos_v, fill_v, ones_v, stage_v, tiny_v,
             pub_v, cnt_sh, deg_sh):
    s = lax.axis_index("s")
    pltpu.sync_copy(src_hbm.at[s], src_v)
    pltpu.sync_copy(dst_hbm.at[s], dst_v)
    for i in range(CHUNK // 16):
        ones_v[pl.ds(i * 16, 16)] = jnp.ones((16,), jnp.float32)
    for i in range(1024 // 16):
        stage_v[pl.ds(i * 16, 16)] = jnp.zeros((16,), jnp.float32)
    @pl.when(s < NP // 1024)
    def _():
        pltpu.sync_copy(stage_v, deg_sh.at[pl.ds(s * 1024, 1024)])

    def count_body(j, n0):
        for k in range(CHUNK // 16):
            d16 = dst_v[j, pl.ds(k * 16, 16)]
            n0 = n0 + jnp.sum(jnp.where(d16 < HALF, 1, 0).astype(jnp.int32))
        return n0

    n0 = lax.fori_loop(0, NCHUNK, count_body, jnp.int32(0))
    n1 = jnp.int32(EPW) - n0

    for rr in range(8):
        pub_v[rr, pl.ds(0, 16)] = jnp.full((16,), n0 if rr == 0 else n1,
                                           jnp.int32)
    pltpu.sync_copy(pub_v, cnt_sh.at[s])

    _fill_pattern(fill_v, 0)
    h = s // 8
    blk = (s % 8) * FILLCH * CHUNK + h * PSLOT
    pltpu.sync_copy(fill_v, srcp_out.at[pl.ds(blk, 16384)])
    pltpu.sync_copy(fill_v.at[pl.ds(0, FILLCH * CHUNK - 16384)],
                    srcp_out.at[pl.ds(blk + 16384, FILLCH * CHUNK - 16384)])
    _fill_pattern(fill_v, HALF)
    pltpu.sync_copy(fill_v, dstp_out.at[pl.ds(blk, 16384)])
    pltpu.sync_copy(fill_v.at[pl.ds(0, FILLCH * CHUNK - 16384)],
                    dstp_out.at[pl.ds(blk + 16384, FILLCH * CHUNK - 16384)])

    plsc.subcore_barrier()

    pref0 = jnp.int32(0)
    pref1 = jnp.int32(0)
    tot0 = jnp.int32(0)
    tot1 = jnp.int32(0)
    for r in range(NS):
        pltpu.sync_copy(cnt_sh.at[r], pub_v)
        c0r = pub_v[0, pl.ds(0, 16)][0]
        c1r = pub_v[1, pl.ds(0, 16)][0]
        take = r < s
        pref0 = pref0 + jnp.where(take, c0r, 0)
        pref1 = pref1 + jnp.where(take, c1r, 0)
        tot0 = tot0 + c0r
        tot1 = tot1 + c1r

    def place_body(j, carry):
        p0, p1 = carry
        for k in range(CHUNK // 16):
            sl = pl.ds(k * 16, 16)
            d16 = dst_v[j, sl]
            m = d16 < HALF
            mi = jnp.where(m, 1, 0).astype(jnp.int32)
            c0 = plsc.cumsum(mi)
            c1 = plsc.cumsum(1 - mi)
            pos = jnp.where(m, p0 + c0 - 1, PSLOT + p1 + c1 - 1)
            pos_v[j, sl] = pos
            dl_v[j, sl] = jnp.where(m, d16, d16 - HALF)
            msum = jnp.sum(mi)
            p0 = p0 + msum
            p1 = p1 + (16 - msum)
        pltpu.sync_copy(ones_v, deg_sh.at[dst_v.at[j]], add=True)
        pltpu.sync_copy(src_v.at[j], srcp_out.at[pos_v.at[j]])
        pltpu.sync_copy(dl_v.at[j], dstp_out.at[pos_v.at[j]])
        return (p0, p1)

    lax.fori_loop(0, NCHUNK, place_body, (pref0, pref1))

    plsc.subcore_barrier()
    @pl.when(s < NP // 1024)
    def _():
        pltpu.sync_copy(deg_sh.at[pl.ds(s * 1024, 1024)], stage_v)
        pltpu.sync_copy(stage_v, deg_out.at[pl.ds(s * 1024, 1024)])

    @pl.when(s == 0)
    def _():
        nch0 = (tot0 + (CHUNK - 1)) // CHUNK
        nch1 = (tot1 + (CHUNK - 1)) // CHUNK
        lane = lax.iota(jnp.int32, 16)
        tiny_v[...] = jnp.where(lane == 0, nch0, jnp.where(lane == 1, nch1, 0))
        pltpu.sync_copy(tiny_v, cnt_out)


@functools.partial(
    pl.kernel,
    out_type=[jax.ShapeDtypeStruct((NP, FH), jnp.float32) for _ in range(2)],
    mesh=_mesh2,
    scratch_types=[
        pltpu.VMEM((CHUNK,), jnp.int32),
        pltpu.VMEM((CHUNK,), jnp.int32),
        pltpu.VMEM((CHUNK,), jnp.int32),
        pltpu.VMEM((CHUNK,), jnp.int32),
        pltpu.VMEM((CHUNK, FH), jnp.float32),
        pltpu.VMEM((CHUNK, FH), jnp.float32),
        pltpu.VMEM((ACCR // NS, FH), jnp.float32),
        pltpu.VMEM_SHARED((ACCR, FH), jnp.float32),
        pltpu.SemaphoreType.DMA,
        pltpu.SemaphoreType.DMA,
    ],
)
def _sc_round(u0_hbm, u1_hbm, srcp_hbm, dstp_hbm, zero_hbm, s0_out, s1_out,
              si_a, si_b, di_a, di_b, buf_a, buf_b, stage_v, acc, sem_a, sem_b):
    c = lax.axis_index("c")
    s = lax.axis_index("s")
    sidx = (si_a, si_b)
    didx = (di_a, di_b)
    bufs = (buf_a, buf_b)
    sems = (sem_a, sem_b)
    zrows = ACCR // NS

    def stage(jl):
        j = s * MAXCH + jl
        pltpu.sync_copy(srcp_hbm.at[c].at[j], sidx[jl % 2])
        pltpu.sync_copy(dstp_hbm.at[c].at[j], didx[jl % 2])

    for u_hbm, s_out in ((u0_hbm, s0_out), (u1_hbm, s1_out)):
        pltpu.sync_copy(zero_hbm, stage_v)
        pltpu.sync_copy(stage_v, acc.at[pl.ds(s * zrows, zrows)])
        plsc.subcore_barrier()
        stage(0)
        desc = {0: pltpu.async_copy(u_hbm.at[sidx[0]], bufs[0], sems[0])}
        for jl in range(MAXCH):
            if jl + 1 < MAXCH:
                stage(jl + 1)
                desc[jl + 1] = pltpu.async_copy(
                    u_hbm.at[sidx[(jl + 1) % 2]], bufs[(jl + 1) % 2],
                    sems[(jl + 1) % 2])
            desc[jl].wait()
            pltpu.sync_copy(bufs[jl % 2], acc.at[didx[jl % 2]], add=True)
        plsc.subcore_barrier()
        r0 = s * (HALF // NS)
        pltpu.sync_copy(acc.at[pl.ds(r0, HALF // NS)],
                        stage_v.at[pl.ds(0, HALF // NS)])
        pltpu.sync_copy(stage_v.at[pl.ds(0, HALF // NS)],
                        s_out.at[pl.ds(c * HALF + r0, HALF // NS)])
        plsc.subcore_barrier()


def _mm_kernel(x_ref, w_ref, b_ref, o_ref, *, act):
    y = jnp.dot(x_ref[...], w_ref[...], preferred_element_type=jnp.float32)
    y = y + b_ref[...]
    if act:
        y = jnp.maximum(y, 0.0)
    o_ref[...] = y


def _tc_linear(x, w, b, act, dout):
    return pl.pallas_call(
        functools.partial(_mm_kernel, act=act),
        grid=(GRID,),
        in_specs=[
            pl.BlockSpec((ROWBLK, DIM), lambda i: (i, 0)),
            pl.BlockSpec((DIM, dout), lambda i: (0, 0)),
            pl.BlockSpec((1, dout), lambda i: (0, 0)),
        ],
        out_specs=pl.BlockSpec((ROWBLK, dout), lambda i: (i, 0)),
        out_shape=jax.ShapeDtypeStruct((NP, dout), jnp.float32),
    )(x, w, b.reshape(1, dout))


def _mlp_kernel(h_ref, w1_ref, b1_ref, w2_ref, b2_ref, dinv_ref, u0_ref, u1_ref):
    t = jnp.maximum(jnp.dot(h_ref[...], w1_ref[...],
                            preferred_element_type=jnp.float32) + b1_ref[...], 0.0)
    e = jnp.dot(t, w2_ref[...], preferred_element_type=jnp.float32) + b2_ref[...]
    u = dinv_ref[...] * e
    u0_ref[...] = u[:, :FH]
    u1_ref[...] = u[:, FH:]


def _tc_mlp_scaled(h, w1, b1, w2, b2, dinv):
    return pl.pallas_call(
        _mlp_kernel,
        grid=(GRID,),
        in_specs=[
            pl.BlockSpec((ROWBLK, DIM), lambda i: (i, 0)),
            pl.BlockSpec((DIM, DIM), lambda i: (0, 0)),
            pl.BlockSpec((1, DIM), lambda i: (0, 0)),
            pl.BlockSpec((DIM, DIM), lambda i: (0, 0)),
            pl.BlockSpec((1, DIM), lambda i: (0, 0)),
            pl.BlockSpec((ROWBLK, 1), lambda i: (i, 0)),
        ],
        out_specs=[pl.BlockSpec((ROWBLK, FH), lambda i: (i, 0))] * 2,
        out_shape=[jax.ShapeDtypeStruct((NP, FH), jnp.float32)] * 2,
    )(h, w1, b1.reshape(1, DIM), w2, b2.reshape(1, DIM), dinv)


def _deg_finish_kernel(dp_ref, dinv_ref, c1_ref, sqd_ref):
    deg = dp_ref[...] + 1.0
    r = lax.broadcasted_iota(jnp.int32, (NP // 128, 128), 0) * 128 \
        + lax.broadcasted_iota(jnp.int32, (NP // 128, 128), 1)
    mask = r < N_REAL
    dinv = jax.lax.rsqrt(deg)
    dinv_ref[...] = jnp.where(mask, dinv, 0.0)
    c1_ref[...] = jnp.where(mask, (1.0 - ALPHA) * dinv * dinv, 0.0)
    sqd_ref[...] = jnp.sqrt(deg)


def _tc_deg_finish(deg):
    return pl.pallas_call(
        _deg_finish_kernel,
        out_shape=[jax.ShapeDtypeStruct((NP // 128, 128), jnp.float32)] * 3,
    )(deg.reshape(NP // 128, 128))


def _comb_kernel(s0_ref, s1_ref, u0_ref, u1_ref, i0_ref, i1_ref, c1_ref,
                 o0_ref, o1_ref):
    c1 = c1_ref[...]
    o0_ref[...] = c1 * (s0_ref[...] + u0_ref[...]) + ALPHA * i0_ref[...]
    o1_ref[...] = c1 * (s1_ref[...] + u1_ref[...]) + ALPHA * i1_ref[...]


def _tc_combine(s0, s1, u0, u1, ui0, ui1, c1):
    fspec = pl.BlockSpec((ROWBLK, FH), lambda i: (i, 0))
    return pl.pallas_call(
        _comb_kernel,
        grid=(GRID,),
        in_specs=[fspec] * 6 + [pl.BlockSpec((ROWBLK, 1), lambda i: (i, 0))],
        out_specs=[fspec] * 2,
        out_shape=[jax.ShapeDtypeStruct((NP, FH), jnp.float32)] * 2,
    )(s0, s1, u0, u1, ui0, ui1, c1)


def _final_kernel(s0_ref, s1_ref, u0_ref, u1_ref, i0_ref, i1_ref, c1_ref,
                  sqd_ref, h_ref, o_ref):
    c1 = c1_ref[...]
    sqd = sqd_ref[...]
    un0 = c1 * (s0_ref[...] + u0_ref[...]) + ALPHA * i0_ref[...]
    un1 = c1 * (s1_ref[...] + u1_ref[...]) + ALPHA * i1_ref[...]
    o_ref[...] = h_ref[...] + jnp.concatenate(
        [jnp.maximum(sqd * un0, 0.0), jnp.maximum(sqd * un1, 0.0)], axis=1)


def _tc_final(s0, s1, u0, u1, ui0, ui1, c1, sqd, h):
    fspec = pl.BlockSpec((ROWBLK, FH), lambda i: (i, 0))
    cspec = pl.BlockSpec((ROWBLK, 1), lambda i: (i, 0))
    return pl.pallas_call(
        _final_kernel,
        grid=(GRID,),
        in_specs=[fspec] * 6 + [cspec, cspec,
                                pl.BlockSpec((ROWBLK, DIM), lambda i: (i, 0))],
        out_specs=pl.BlockSpec((ROWBLK, DIM), lambda i: (i, 0)),
        out_shape=jax.ShapeDtypeStruct((NP, DIM), jnp.float32),
    )(s0, s1, u0, u1, ui0, ui1, c1, sqd, h)


def kernel(x, edge_index, W_pre, b_pre, W1_0, b1_0, W2_0, b2_0,
           W1_1, b1_1, W2_1, b2_1, W_head, b_head):
    n = x.shape[0]
    src = edge_index[0].astype(jnp.int32)
    dst = edge_index[1].astype(jnp.int32)
    e_real = src.shape[0]
    epad = NS * EPW - e_real
    pid = jnp.arange(epad, dtype=jnp.int32)
    src_c = jnp.concatenate([src, (pid * 37) % n]).reshape(NS, NCHUNK, CHUNK)
    dst_c = jnp.concatenate([dst, n + pid % (NP - n)]).reshape(NS, NCHUNK, CHUNK)

    zero_stage = jnp.zeros((ACCR // NS, FH), jnp.float32)
    xp = jnp.pad(x, ((0, NP - n), (0, 0)))

    deg, srcp, dstp, _cnt = _sc_prep(src_c, dst_c)
    srcp = srcp.reshape(2, PCH, CHUNK)
    dstp = dstp.reshape(2, PCH, CHUNK)
    dinv, c1, sqd = _tc_deg_finish(deg)
    dinv = dinv.reshape(NP, 1)
    c1 = c1.reshape(NP, 1)
    sqd = sqd.reshape(NP, 1)

    h = _tc_linear(xp, W_pre, b_pre, True, DIM)
    for (w1, b1, w2, b2) in ((W1_0, b1_0, W2_0, b2_0), (W1_1, b1_1, W2_1, b2_1)):
        ui0, ui1 = _tc_mlp_scaled(h, w1, b1, w2, b2, dinv)
        u0, u1 = ui0, ui1
        for it in range(K_ITER):
            s0, s1 = _sc_round(u0, u1, srcp, dstp, zero_stage)
            if it + 1 < K_ITER:
                u0, u1 = _tc_combine(s0, s1, u0, u1, ui0, ui1, c1)
            else:
                h = _tc_final(s0, s1, u0, u1, ui0, ui1, c1, sqd, h)

    out = _tc_linear(h, W_head, b_head, False, 64)
    return out[:n]
